# Initial kernel scaffold; baseline (speedup 1.0000x reference)
import jax
import jax.numpy as jnp
from jax.experimental import pallas as pl


def kernel(pos_u, pos_v, neg_v, batch_size, u_emb, v_emb):
    raise NotImplementedError("write your pallas kernel here")


# speedup vs baseline: 4.1165x; 4.1165x over previous
import functools

import jax
import jax.numpy as jnp
from jax import lax
from jax.experimental import pallas as pl
from jax.experimental.pallas import tpu as pltpu
from jax.experimental.pallas import tpu_sc as plsc

V = 1000000
D = 64
B = 16384
NNEG = 20

NC = 2
NS = 16
NW = NC * NS
BW = B // NW
CHUNK = 16
NCH = BW // CHUNK
NEGC = CHUNK * NNEG
_NEG_PIECES = ((0, 128), (128, 128), (256, 64))


def _fire_neg(v_hbm, idx_n, buf, sem, c):
    for off, cnt in _NEG_PIECES:
        pltpu.make_async_copy(
            v_hbm.at[idx_n.at[pl.ds(c * NEGC + off, cnt)]],
            buf.at[pl.ds(off, cnt)],
            sem,
        ).start()


def _wait_neg(v_hbm, idx_n, buf, sem):
    for off, cnt in _NEG_PIECES:
        pltpu.make_async_copy(
            v_hbm.at[idx_n.at[pl.ds(off, cnt)]],
            buf.at[pl.ds(off, cnt)],
            sem,
        ).wait()


def _compute_chunk(c, buf, urows, vrows, score_loc, neg_loc):
    iota = lax.iota(jnp.int32, CHUNK)
    row_uv = iota + c * CHUNK
    row_n = [iota * NNEG + n for n in range(NNEG)]
    zero = jnp.zeros((CHUNK,), jnp.float32)

    def body(d, accs):
        col = jnp.full((CHUNK,), d, jnp.int32)
        u_col = plsc.load_gather(urows, [row_uv, col])
        v_col = plsc.load_gather(vrows, [row_uv, col])
        new = [accs[0] + u_col * v_col]
        for n in range(NNEG):
            n_col = plsc.load_gather(buf, [row_n[n], col])
            new.append(accs[n + 1] + n_col * u_col)
        return tuple(new)

    accs = lax.fori_loop(0, D, body, (zero,) * (NNEG + 1))
    score_loc[pl.ds(c * CHUNK, CHUNK)] = accs[0]
    for n in range(NNEG):
        neg_loc[pl.ds(c * NEGC + n * CHUNK, CHUNK)] = accs[n + 1]


def _sc_body(u_hbm, v_hbm, pos_u, pos_v, neg_flat,
             score_out, neg_out,
             idx_u, idx_v, idx_n, urows, vrows, nrows,
             score_loc, neg_loc, sem_uv, sem_n0, sem_n1):
    wid = lax.axis_index("s") * NC + lax.axis_index("c")
    base = wid * BW

    pltpu.sync_copy(pos_u.at[pl.ds(base, BW)], idx_u)
    pltpu.sync_copy(pos_v.at[pl.ds(base, BW)], idx_v)
    pltpu.sync_copy(neg_flat.at[pl.ds(base * NNEG, BW * NNEG)], idx_n)

    for k in range(BW // 128):
        pltpu.make_async_copy(
            u_hbm.at[idx_u.at[pl.ds(k * 128, 128)]],
            urows.at[pl.ds(k * 128, 128)], sem_uv).start()
        pltpu.make_async_copy(
            v_hbm.at[idx_v.at[pl.ds(k * 128, 128)]],
            vrows.at[pl.ds(k * 128, 128)], sem_uv).start()

    _fire_neg(v_hbm, idx_n, nrows.at[0], sem_n0, 0)
    _fire_neg(v_hbm, idx_n, nrows.at[1], sem_n1, 1)

    for k in range(BW // 128):
        pltpu.make_async_copy(
            u_hbm.at[idx_u.at[pl.ds(k * 128, 128)]],
            urows.at[pl.ds(k * 128, 128)], sem_uv).wait()
        pltpu.make_async_copy(
            v_hbm.at[idx_v.at[pl.ds(k * 128, 128)]],
            vrows.at[pl.ds(k * 128, 128)], sem_uv).wait()

    @pl.loop(0, NCH // 2)
    def _chunks(g):
        c0 = g * 2
        _wait_neg(v_hbm, idx_n, nrows.at[0], sem_n0)
        _compute_chunk(c0, nrows.at[0], urows, vrows, score_loc, neg_loc)

        @pl.when(g < NCH // 2 - 1)
        def _():
            _fire_neg(v_hbm, idx_n, nrows.at[0], sem_n0, c0 + 2)

        _wait_neg(v_hbm, idx_n, nrows.at[1], sem_n1)
        _compute_chunk(c0 + 1, nrows.at[1], urows, vrows, score_loc, neg_loc)

        @pl.when(g < NCH // 2 - 1)
        def _():
            _fire_neg(v_hbm, idx_n, nrows.at[1], sem_n1, c0 + 3)

    pltpu.sync_copy(score_loc, score_out.at[pl.ds(base, BW)])
    pltpu.sync_copy(neg_loc, neg_out.at[pl.ds(base * NNEG, BW * NNEG)])


def _loss_body(inv_b_ref, s_ref, n_ref, o_ref):
    def logsig(x):
        return jnp.minimum(x, 0.0) - jnp.log1p(jnp.exp(-jnp.abs(x)))

    total = jnp.sum(logsig(s_ref[...])) + jnp.sum(logsig(-n_ref[...]))
    o_ref[0, 0] = -total * inv_b_ref[0]


@jax.jit
def _run(pos_u, pos_v, neg_v, u_emb, v_emb, inv_b):
    mesh = plsc.VectorSubcoreMesh(
        core_axis_name="c", subcore_axis_name="s",
        num_cores=NC, num_subcores=NS)
    sc = pl.kernel(
        _sc_body,
        out_type=(
            jax.ShapeDtypeStruct((B,), jnp.float32),
            jax.ShapeDtypeStruct((B * NNEG,), jnp.float32),
        ),
        mesh=mesh,
        compiler_params=pltpu.CompilerParams(
            needs_layout_passes=False, use_tc_tiling_on_sc=False),
        scratch_types=[
            pltpu.VMEM((BW,), jnp.int32),
            pltpu.VMEM((BW,), jnp.int32),
            pltpu.VMEM((BW * NNEG,), jnp.int32),
            pltpu.VMEM((BW, D), jnp.float32),
            pltpu.VMEM((BW, D), jnp.float32),
            pltpu.VMEM((2, NEGC, D), jnp.float32),
            pltpu.VMEM((BW,), jnp.float32),
            pltpu.VMEM((BW * NNEG,), jnp.float32),
            pltpu.SemaphoreType.DMA,
            pltpu.SemaphoreType.DMA,
            pltpu.SemaphoreType.DMA,
        ],
    )
    score, neg_score = sc(u_emb, v_emb, pos_u, pos_v, neg_v.reshape(-1))

    loss = pl.pallas_call(
        _loss_body,
        out_shape=jax.ShapeDtypeStruct((1, 1), jnp.float32),
        in_specs=[
            pl.BlockSpec(memory_space=pltpu.SMEM),
            pl.BlockSpec(memory_space=pltpu.VMEM),
            pl.BlockSpec(memory_space=pltpu.VMEM),
        ],
        out_specs=pl.BlockSpec(memory_space=pltpu.SMEM),
    )(inv_b, score.reshape(B // 128, 128), neg_score.reshape(B * NNEG // 128, 128))
    return loss[0, 0]


def kernel(pos_u, pos_v, neg_v, batch_size, u_emb, v_emb):
    pos_u = jnp.asarray(pos_u, jnp.int32)
    pos_v = jnp.asarray(pos_v, jnp.int32)
    neg_v = jnp.asarray(neg_v, jnp.int32)
    inv_b = jnp.full((1,), 1.0 / batch_size, jnp.float32)
    return _run(pos_u, pos_v, neg_v, u_emb, v_emb, inv_b)


# speedup vs baseline: 4.1832x; 1.0162x over previous
import functools

import jax
import jax.numpy as jnp
from jax import lax
from jax.experimental import pallas as pl
from jax.experimental.pallas import tpu as pltpu
from jax.experimental.pallas import tpu_sc as plsc

V = 1000000
D = 64
B = 16384
NNEG = 20

NC = 2
NS = 16
NW = NC * NS
BW = B // NW
CHUNK = 16
NCH = BW // CHUNK
NEGC = CHUNK * NNEG
_NEG_PIECES = ((0, 128), (128, 128), (256, 64))


def _fire_neg(v_hbm, idx_n, buf, sem, c):
    for off, cnt in _NEG_PIECES:
        pltpu.make_async_copy(
            v_hbm.at[idx_n.at[pl.ds(c * NEGC + off, cnt)]],
            buf.at[pl.ds(off, cnt)],
            sem,
        ).start()


def _wait_neg(v_hbm, idx_n, buf, sem):
    for off, cnt in _NEG_PIECES:
        pltpu.make_async_copy(
            v_hbm.at[idx_n.at[pl.ds(off, cnt)]],
            buf.at[pl.ds(off, cnt)],
            sem,
        ).wait()


def _compute_chunk(c, buf, urows, vrows, score_loc, neg_loc):
    iota = lax.iota(jnp.int32, CHUNK)
    row_uv = iota + c * CHUNK
    row_n0 = iota * NNEG
    zero = jnp.zeros((CHUNK,), jnp.float32)

    def body(d, accs):
        col = jnp.full((CHUNK,), d, jnp.int32)
        u_col = plsc.load_gather(urows, [row_uv, col])
        v_col = plsc.load_gather(vrows, [row_uv, col])
        new = [accs[0] + u_col * v_col]
        for n in range(NNEG):
            n_col = plsc.load_gather(buf, [row_n0 + n, col])
            new.append(accs[n + 1] + n_col * u_col)
        return tuple(new)

    accs = lax.fori_loop(0, D, body, (zero,) * (NNEG + 1), unroll=2)
    score_loc[pl.ds(c * CHUNK, CHUNK)] = accs[0]
    for n in range(NNEG):
        neg_loc[pl.ds(c * NEGC + n * CHUNK, CHUNK)] = accs[n + 1]


def _sc_body(u_hbm, v_hbm, pos_u, pos_v, neg_flat,
             score_out, neg_out,
             idx_u, idx_v, idx_n, urows, vrows, nrows,
             score_loc, neg_loc, sem_uv, sem_n0, sem_n1):
    wid = lax.axis_index("s") * NC + lax.axis_index("c")
    base = wid * BW

    pltpu.sync_copy(pos_u.at[pl.ds(base, BW)], idx_u)
    pltpu.sync_copy(pos_v.at[pl.ds(base, BW)], idx_v)
    pltpu.sync_copy(neg_flat.at[pl.ds(base * NNEG, BW * NNEG)], idx_n)

    for k in range(BW // 128):
        pltpu.make_async_copy(
            u_hbm.at[idx_u.at[pl.ds(k * 128, 128)]],
            urows.at[pl.ds(k * 128, 128)], sem_uv).start()
        pltpu.make_async_copy(
            v_hbm.at[idx_v.at[pl.ds(k * 128, 128)]],
            vrows.at[pl.ds(k * 128, 128)], sem_uv).start()

    _fire_neg(v_hbm, idx_n, nrows.at[0], sem_n0, 0)
    _fire_neg(v_hbm, idx_n, nrows.at[1], sem_n1, 1)

    for k in range(BW // 128):
        pltpu.make_async_copy(
            u_hbm.at[idx_u.at[pl.ds(k * 128, 128)]],
            urows.at[pl.ds(k * 128, 128)], sem_uv).wait()
        pltpu.make_async_copy(
            v_hbm.at[idx_v.at[pl.ds(k * 128, 128)]],
            vrows.at[pl.ds(k * 128, 128)], sem_uv).wait()

    @pl.loop(0, NCH // 2)
    def _chunks(g):
        c0 = g * 2
        _wait_neg(v_hbm, idx_n, nrows.at[0], sem_n0)
        _compute_chunk(c0, nrows.at[0], urows, vrows, score_loc, neg_loc)

        @pl.when(g < NCH // 2 - 1)
        def _():
            _fire_neg(v_hbm, idx_n, nrows.at[0], sem_n0, c0 + 2)

        _wait_neg(v_hbm, idx_n, nrows.at[1], sem_n1)
        _compute_chunk(c0 + 1, nrows.at[1], urows, vrows, score_loc, neg_loc)

        @pl.when(g < NCH // 2 - 1)
        def _():
            _fire_neg(v_hbm, idx_n, nrows.at[1], sem_n1, c0 + 3)

    pltpu.sync_copy(score_loc, score_out.at[pl.ds(base, BW)])
    pltpu.sync_copy(neg_loc, neg_out.at[pl.ds(base * NNEG, BW * NNEG)])


def _loss_body(inv_b_ref, s_ref, n_ref, o_ref):
    def logsig(x):
        return jnp.minimum(x, 0.0) - jnp.log1p(jnp.exp(-jnp.abs(x)))

    total = jnp.sum(logsig(s_ref[...])) + jnp.sum(logsig(-n_ref[...]))
    o_ref[0, 0] = -total * inv_b_ref[0]


@jax.jit
def _run(pos_u, pos_v, neg_v, u_emb, v_emb, inv_b):
    mesh = plsc.VectorSubcoreMesh(
        core_axis_name="c", subcore_axis_name="s",
        num_cores=NC, num_subcores=NS)
    sc = pl.kernel(
        _sc_body,
        out_type=(
            jax.ShapeDtypeStruct((B,), jnp.float32),
            jax.ShapeDtypeStruct((B * NNEG,), jnp.float32),
        ),
        mesh=mesh,
        compiler_params=pltpu.CompilerParams(
            needs_layout_passes=False, use_tc_tiling_on_sc=False),
        scratch_types=[
            pltpu.VMEM((BW,), jnp.int32),
            pltpu.VMEM((BW,), jnp.int32),
            pltpu.VMEM((BW * NNEG,), jnp.int32),
            pltpu.VMEM((BW, D), jnp.float32),
            pltpu.VMEM((BW, D), jnp.float32),
            pltpu.VMEM((2, NEGC, D), jnp.float32),
            pltpu.VMEM((BW,), jnp.float32),
            pltpu.VMEM((BW * NNEG,), jnp.float32),
            pltpu.SemaphoreType.DMA,
            pltpu.SemaphoreType.DMA,
            pltpu.SemaphoreType.DMA,
        ],
    )
    score, neg_score = sc(u_emb, v_emb, pos_u, pos_v, neg_v.reshape(-1))

    loss = pl.pallas_call(
        _loss_body,
        out_shape=jax.ShapeDtypeStruct((1, 1), jnp.float32),
        in_specs=[
            pl.BlockSpec(memory_space=pltpu.SMEM),
            pl.BlockSpec(memory_space=pltpu.VMEM),
            pl.BlockSpec(memory_space=pltpu.VMEM),
        ],
        out_specs=pl.BlockSpec(memory_space=pltpu.SMEM),
    )(inv_b, score.reshape(B // 128, 128), neg_score.reshape(B * NNEG // 128, 128))
    return loss[0, 0]


def kernel(pos_u, pos_v, neg_v, batch_size, u_emb, v_emb):
    pos_u = jnp.asarray(pos_u, jnp.int32)
    pos_v = jnp.asarray(pos_v, jnp.int32)
    neg_v = jnp.asarray(neg_v, jnp.int32)
    inv_b = jnp.full((1,), 1.0 / batch_size, jnp.float32)
    return _run(pos_u, pos_v, neg_v, u_emb, v_emb, inv_b)


# speedup vs baseline: 14.4716x; 3.4595x over previous
import jax
import jax.numpy as jnp
from jax import lax
from jax.experimental import pallas as pl
from jax.experimental.pallas import tpu as pltpu
from jax.experimental.pallas import tpu_sc as plsc

V = 1000000
D = 64
B = 16384
NNEG = 20

NC = 2
NS = 16
NW = NC * NS
BW = B // NW
CHUNK = 16
NCH = BW // CHUNK
NEGC = CHUNK * NNEG
_NEG_PIECES = ((0, 128), (128, 128), (256, 64))


def _fire_neg(v_hbm, idx_n, buf, sem, c):
    for off, cnt in _NEG_PIECES:
        pltpu.make_async_copy(
            v_hbm.at[idx_n.at[pl.ds(c * NEGC + off, cnt)]],
            buf.at[pl.ds(off, cnt), pl.ds(0, D)],
            sem,
        ).start()


def _wait_neg(v_hbm, idx_n, buf, sem):
    for off, cnt in _NEG_PIECES:
        pltpu.make_async_copy(
            v_hbm.at[idx_n.at[pl.ds(off, cnt)]],
            buf.at[pl.ds(off, cnt), pl.ds(0, D)],
            sem,
        ).wait()


def _compute_chunk(c, buf, urows, vrows, score_loc, neg_loc):
    iota = lax.iota(jnp.int32, CHUNK)
    row_uv = iota + c * CHUNK
    row_n0 = iota * NNEG
    zero = jnp.zeros((CHUNK,), jnp.float32)

    def body(d, accs):
        col = (iota + d) & (D - 1)
        u_col = plsc.load_gather(urows, [row_uv, col])
        v_col = plsc.load_gather(vrows, [row_uv, col])
        new = [accs[0] + u_col * v_col]
        for n in range(NNEG):
            n_col = plsc.load_gather(buf, [row_n0 + n, col])
            new.append(accs[n + 1] + n_col * u_col)
        return tuple(new)

    accs = lax.fori_loop(0, D, body, (zero,) * (NNEG + 1), unroll=2)
    score_loc[pl.ds(c * CHUNK, CHUNK)] = accs[0]
    for n in range(NNEG):
        neg_loc[pl.ds(c * NEGC + n * CHUNK, CHUNK)] = accs[n + 1]


def _permute_idx(ref, nvec):
    @pl.loop(0, nvec)
    def _(k):
        r = ref[pl.ds(k * CHUNK, CHUNK)]
        s = ((r & jnp.int32(-_TW)) | ((r & (_TW // 2 - 1)) << 1)
             | ((r >> _HS) & 1))
        ref[pl.ds(k * CHUNK, CHUNK)] = s


def _sc_body(u_hbm, v_hbm, pos_u, pos_v, neg_flat,
             score_out, neg_out,
             idx_u, idx_v, idx_n, urows, vrows, nrows,
             score_loc, neg_loc, sem_uv, sem_n0, sem_n1):
    wid = lax.axis_index("s") * NC + lax.axis_index("c")
    base = wid * BW

    pltpu.sync_copy(pos_u.at[pl.ds(base, BW)], idx_u)
    pltpu.sync_copy(pos_v.at[pl.ds(base, BW)], idx_v)
    pltpu.sync_copy(neg_flat.at[pl.ds(base * NNEG, BW * NNEG)], idx_n)
    _permute_idx(idx_u, BW // CHUNK)
    _permute_idx(idx_v, BW // CHUNK)
    _permute_idx(idx_n, BW * NNEG // CHUNK)

    for k in range(BW // 128):
        pltpu.make_async_copy(
            u_hbm.at[idx_u.at[pl.ds(k * 128, 128)]],
            urows.at[pl.ds(k * 128, 128), pl.ds(0, D)], sem_uv).start()
        pltpu.make_async_copy(
            v_hbm.at[idx_v.at[pl.ds(k * 128, 128)]],
            vrows.at[pl.ds(k * 128, 128), pl.ds(0, D)], sem_uv).start()

    _fire_neg(v_hbm, idx_n, nrows.at[0], sem_n0, 0)
    _fire_neg(v_hbm, idx_n, nrows.at[1], sem_n1, 1)

    for k in range(BW // 128):
        pltpu.make_async_copy(
            u_hbm.at[idx_u.at[pl.ds(k * 128, 128)]],
            urows.at[pl.ds(k * 128, 128), pl.ds(0, D)], sem_uv).wait()
        pltpu.make_async_copy(
            v_hbm.at[idx_v.at[pl.ds(k * 128, 128)]],
            vrows.at[pl.ds(k * 128, 128), pl.ds(0, D)], sem_uv).wait()

    @pl.loop(0, NCH // 2)
    def _chunks(g):
        c0 = g * 2
        _wait_neg(v_hbm, idx_n, nrows.at[0], sem_n0)
        _compute_chunk(c0, nrows.at[0], urows, vrows, score_loc, neg_loc)

        @pl.when(g < NCH // 2 - 1)
        def _():
            _fire_neg(v_hbm, idx_n, nrows.at[0], sem_n0, c0 + 2)

        _wait_neg(v_hbm, idx_n, nrows.at[1], sem_n1)
        _compute_chunk(c0 + 1, nrows.at[1], urows, vrows, score_loc, neg_loc)

        @pl.when(g < NCH // 2 - 1)
        def _():
            _fire_neg(v_hbm, idx_n, nrows.at[1], sem_n1, c0 + 3)

    pltpu.sync_copy(score_loc, score_out.at[pl.ds(base, BW)])
    pltpu.sync_copy(neg_loc, neg_out.at[pl.ds(base * NNEG, BW * NNEG)])


_TW = 16384
_TG = (V + _TW - 1) // _TW
VS = _TG * _TW
_HS = (_TW // 2).bit_length() - 1


def _transpose_body(ut_ref, vt_ref, u_out, v_out):
    eye = (lax.broadcasted_iota(jnp.int32, (2 * D, 2 * D), 0)
           == lax.broadcasted_iota(jnp.int32, (2 * D, 2 * D), 1)
           ).astype(jnp.float32)
    dn = (((0,), (0,)), ((), ()))
    h = _TW // 2

    def tr(ref, out):
        x = ref[...]
        xs = jnp.concatenate([x[:, :h], x[:, h:]], axis=0)
        out[...] = lax.dot_general(xs, eye, dn,
                                   preferred_element_type=jnp.float32)

    tr(ut_ref, u_out)
    tr(vt_ref, v_out)


def _relinearize(u_emb, v_emb):
    return pl.pallas_call(
        _transpose_body,
        grid=(_TG,),
        in_specs=[
            pl.BlockSpec((D, _TW), lambda i: (0, i)),
            pl.BlockSpec((D, _TW), lambda i: (0, i)),
        ],
        out_specs=[
            pl.BlockSpec((_TW // 2, 2 * D), lambda i: (i, 0)),
            pl.BlockSpec((_TW // 2, 2 * D), lambda i: (i, 0)),
        ],
        out_shape=[
            jax.ShapeDtypeStruct((VS // 2, 2 * D), jnp.float32),
            jax.ShapeDtypeStruct((VS // 2, 2 * D), jnp.float32),
        ],
    )(u_emb.T, v_emb.T)


def _loss_body(inv_b_ref, s_ref, n_ref, o_ref):
    def logsig(x):
        return jnp.minimum(x, 0.0) - jnp.log1p(jnp.exp(-jnp.abs(x)))

    total = jnp.sum(logsig(s_ref[...])) + jnp.sum(logsig(-n_ref[...]))
    o_ref[0, 0] = -total * inv_b_ref[0]


@jax.jit
def _run(pos_u, pos_v, neg_v, u_emb, v_emb, inv_b):
    mesh = plsc.VectorSubcoreMesh(
        core_axis_name="c", subcore_axis_name="s",
        num_cores=NC, num_subcores=NS)
    sc = pl.kernel(
        _sc_body,
        out_type=(
            jax.ShapeDtypeStruct((B,), jnp.float32),
            jax.ShapeDtypeStruct((B * NNEG,), jnp.float32),
        ),
        mesh=mesh,
        compiler_params=pltpu.CompilerParams(
            needs_layout_passes=False, use_tc_tiling_on_sc=False),
        scratch_types=[
            pltpu.VMEM((BW,), jnp.int32),
            pltpu.VMEM((BW,), jnp.int32),
            pltpu.VMEM((BW * NNEG,), jnp.int32),
            pltpu.VMEM((BW, D), jnp.float32),
            pltpu.VMEM((BW, D), jnp.float32),
            pltpu.VMEM((2, NEGC, D), jnp.float32),
            pltpu.VMEM((BW,), jnp.float32),
            pltpu.VMEM((BW * NNEG,), jnp.float32),
            pltpu.SemaphoreType.DMA,
            pltpu.SemaphoreType.DMA,
            pltpu.SemaphoreType.DMA,
        ],
    )
    u_lin, v_lin = _relinearize(u_emb, v_emb)
    score, neg_score = sc(u_lin.reshape(VS, D), v_lin.reshape(VS, D),
                          pos_u, pos_v, neg_v.reshape(-1))

    loss = pl.pallas_call(
        _loss_body,
        out_shape=jax.ShapeDtypeStruct((1, 1), jnp.float32),
        in_specs=[
            pl.BlockSpec(memory_space=pltpu.SMEM),
            pl.BlockSpec(memory_space=pltpu.VMEM),
            pl.BlockSpec(memory_space=pltpu.VMEM),
        ],
        out_specs=pl.BlockSpec(memory_space=pltpu.SMEM),
    )(inv_b, score.reshape(B // 128, 128), neg_score.reshape(B * NNEG // 128, 128))
    return loss[0, 0]


def kernel(pos_u, pos_v, neg_v, batch_size, u_emb, v_emb):
    pos_u = jnp.asarray(pos_u, jnp.int32)
    pos_v = jnp.asarray(pos_v, jnp.int32)
    neg_v = jnp.asarray(neg_v, jnp.int32)
    inv_b = jnp.full((1,), 1.0 / batch_size, jnp.float32)
    return _run(pos_u, pos_v, neg_v, u_emb, v_emb, inv_b)
